# baseline (speedup 1.0000x reference)
import functools

import jax
import jax.numpy as jnp
from jax import lax
from jax.experimental import pallas as pl
from jax.experimental.pallas import tpu as pltpu
from jax.experimental.pallas import tpu_sc as plsc

_B = 1024
_NPB = 62
_N = _B * _NPB
_E = 317440
_NW = 32
_EPW = _E // _NW
_EH = _EPW // 2
_NCHUNK = 16
_CHB = 4096
_CAP = 2048
_S2CAP = 128
_EPS = 1e-5


def _sc_mesh():
    return plsc.VectorSubcoreMesh(core_axis_name="c", subcore_axis_name="s",
                                  num_cores=2, num_subcores=16)


def _wid():
    return lax.axis_index("s") * 2 + lax.axis_index("c")


def _zero_ref(ref, n, dtype):
    z = jnp.zeros((16,), dtype)

    def body(i, _):
        ref[pl.ds(i * 16, 16)] = z
        return 0

    lax.fori_loop(0, n // 16, body, 0)


def _pa_body(src_hbm, dst_hbm, ew_hbm,
             bsrc_hbm, bdl_hbm, bew_hbm, cnts_hbm, degp_hbm,
             bsrc2_hbm, bdl2_hbm, bew2_hbm, cnts2_hbm,
             es_v, ed_v, ee_v, bsrc_v, bdl_v, bew_v, dacc_v, cnt_v,
             ss_v, sd_v, sw_v, cnt2_v):
    wid = _wid()
    iota = lax.iota(jnp.int32, 16)
    zi = jnp.zeros((16,), jnp.int32)
    zf = jnp.zeros((16,), jnp.float32)

    def zb(i, _):
        bsrc_v[pl.ds(i * 16, 16)] = zi
        bdl_v[pl.ds(i * 16, 16)] = zi
        bew_v[pl.ds(i * 16, 16)] = zf
        return 0

    lax.fori_loop(0, _NCHUNK * _CAP // 16, zb, 0)

    base = wid * _EPW
    curs = tuple(jnp.int32(0) for _ in range(_NCHUNK))
    for h in range(2):
        pltpu.sync_copy(src_hbm.at[pl.ds(base + h * _EH, _EH)], es_v)
        pltpu.sync_copy(dst_hbm.at[pl.ds(base + h * _EH, _EH)], ed_v)
        pltpu.sync_copy(ew_hbm.at[pl.ds(base + h * _EH, _EH)], ee_v)

        def vbody(i, cs):
            s16 = es_v[pl.ds(i * 16, 16)]
            d16 = ed_v[pl.ds(i * 16, 16)]
            w16 = ee_v[pl.ds(i * 16, 16)]
            ch = lax.shift_right_logical(d16, 12)
            dl = lax.bitwise_and(d16, 4095)
            new = []
            for c in range(_NCHUNK):
                m = ch == c
                pc = jnp.sum(m.astype(jnp.int32))
                cur = jnp.minimum(cs[c], _CAP - 16)
                off = cur + c * _CAP
                plsc.store_compressed(bsrc_v.at[pl.ds(off, 16)], s16,
                                      mask=m)
                plsc.store_compressed(bdl_v.at[pl.ds(off, 16)], dl,
                                      mask=m)
                plsc.store_compressed(bew_v.at[pl.ds(off, 16)], w16,
                                      mask=m)
                new.append(cur + pc)
            return tuple(new)

        curs = lax.fori_loop(0, _EH // 16, vbody, curs)

    cvec = jnp.zeros((16,), jnp.int32)
    for c in range(_NCHUNK):
        cvec = cvec + jnp.where(iota == c, curs[c], 0)
    cnt_v[...] = cvec
    pltpu.sync_copy(cnt_v, cnts_hbm.at[wid])
    pltpu.sync_copy(bsrc_v, bsrc_hbm.at[wid])
    pltpu.sync_copy(bdl_v, bdl_hbm.at[wid])
    pltpu.sync_copy(bew_v, bew_hbm.at[wid])

    for c in range(_NCHUNK):
        _zero_ref(dacc_v, _CHB, jnp.float32)

        def dbody(i, _):
            dl16 = bdl_v[pl.ds(c * _CAP + i * 16, 16)]
            w16 = bew_v[pl.ds(c * _CAP + i * 16, 16)]
            plsc.addupdate_scatter(dacc_v, [dl16], w16)
            return 0

        lax.fori_loop(0, _CAP // 16, dbody, 0)
        pltpu.sync_copy(dacc_v, degp_hbm.at[wid, c])

    zi16 = jnp.zeros((16,), jnp.int32)
    zf16 = jnp.zeros((16,), jnp.float32)

    def s2_chunk(c, _):
        def z2(i, _2):
            ss_v[pl.ds(i * 16, 16)] = zi16
            sd_v[pl.ds(i * 16, 16)] = zi16
            sw_v[pl.ds(i * 16, 16)] = zf16
            return 0

        lax.fori_loop(0, _S2CAP * 16 // 16, z2, 0)
        cnt_c = jnp.max(jnp.where(iota == c, cvec, 0))
        nv = (cnt_c + 15) // 16

        def s2v(i, cs2):
            base_e = c * _CAP + i * 16
            s16 = bsrc_v[pl.ds(base_e, 16)]
            d16 = bdl_v[pl.ds(base_e, 16)]
            w16 = bew_v[pl.ds(base_e, 16)]
            valid = (iota + i * 16) < cnt_c
            sb = lax.shift_right_logical(d16, 8)
            new = []
            for s in range(16):
                m = (sb == s) & valid
                pc = jnp.sum(m.astype(jnp.int32))
                cur = jnp.minimum(cs2[s], _S2CAP - 16)
                off = s * _S2CAP + cur
                plsc.store_compressed(ss_v.at[pl.ds(off, 16)], s16, mask=m)
                plsc.store_compressed(sd_v.at[pl.ds(off, 16)], d16, mask=m)
                plsc.store_compressed(sw_v.at[pl.ds(off, 16)], w16, mask=m)
                new.append(cur + pc)
            return tuple(new)

        cs2 = lax.fori_loop(0, nv, s2v,
                            tuple(jnp.int32(0) for _ in range(16)))
        c2v = jnp.zeros((16,), jnp.int32)
        for s in range(16):
            c2v = c2v + jnp.where(iota == s, cs2[s], 0)
        cnt2_v[...] = c2v
        pltpu.sync_copy(cnt2_v, cnts2_hbm.at[c, wid])
        for s in range(16):
            pltpu.sync_copy(ss_v.at[pl.ds(s * _S2CAP, _S2CAP)],
                            bsrc2_hbm.at[c, s, wid])
            pltpu.sync_copy(sd_v.at[pl.ds(s * _S2CAP, _S2CAP)],
                            bdl2_hbm.at[c, s, wid])
            pltpu.sync_copy(sw_v.at[pl.ds(s * _S2CAP, _S2CAP)],
                            bew2_hbm.at[c, s, wid])
        return 0

    lax.fori_loop(0, _NCHUNK, s2_chunk, 0)


def _run_pa(src, dst, ew):
    f = pl.kernel(
        _pa_body,
        out_type=[
            jax.ShapeDtypeStruct((_NW, _NCHUNK * _CAP), jnp.int32),
            jax.ShapeDtypeStruct((_NW, _NCHUNK * _CAP), jnp.int32),
            jax.ShapeDtypeStruct((_NW, _NCHUNK * _CAP), jnp.float32),
            jax.ShapeDtypeStruct((_NW, 16), jnp.int32),
            jax.ShapeDtypeStruct((_NW, _NCHUNK, _CHB), jnp.float32),
            jax.ShapeDtypeStruct((_NCHUNK, 16, _NW, _S2CAP), jnp.int32),
            jax.ShapeDtypeStruct((_NCHUNK, 16, _NW, _S2CAP), jnp.int32),
            jax.ShapeDtypeStruct((_NCHUNK, 16, _NW, _S2CAP), jnp.float32),
            jax.ShapeDtypeStruct((_NCHUNK, _NW, 16), jnp.int32),
        ],
        mesh=_sc_mesh(),
        compiler_params=pltpu.CompilerParams(needs_layout_passes=False),
        scratch_types=[
            pltpu.VMEM((_EH,), jnp.int32),
            pltpu.VMEM((_EH,), jnp.int32),
            pltpu.VMEM((_EH,), jnp.float32),
            pltpu.VMEM((_NCHUNK * _CAP,), jnp.int32),
            pltpu.VMEM((_NCHUNK * _CAP,), jnp.int32),
            pltpu.VMEM((_NCHUNK * _CAP,), jnp.float32),
            pltpu.VMEM((_CHB,), jnp.float32),
            pltpu.VMEM((16,), jnp.int32),
            pltpu.VMEM((_S2CAP * 16,), jnp.int32),
            pltpu.VMEM((_S2CAP * 16,), jnp.int32),
            pltpu.VMEM((_S2CAP * 16,), jnp.float32),
            pltpu.VMEM((16,), jnp.int32),
        ],
    )
    return f(src, dst, ew)


def _pb_body(y_hbm, bsrc_hbm, bdl_hbm, bew_hbm, pyp_hbm,
             y_v, src_v, dl_v, ew_v, yacc_v):
    wid = _wid()
    pltpu.sync_copy(y_hbm, y_v)
    for c in range(_NCHUNK):
        _zero_ref(yacc_v, _CHB, jnp.float32)
        pltpu.sync_copy(bsrc_hbm.at[wid, c], src_v)
        pltpu.sync_copy(bdl_hbm.at[wid, c], dl_v)
        pltpu.sync_copy(bew_hbm.at[wid, c], ew_v)

        def body(i, _):
            s16 = src_v[pl.ds(i * 16, 16)]
            dl16 = dl_v[pl.ds(i * 16, 16)]
            w16 = ew_v[pl.ds(i * 16, 16)]
            vals = plsc.load_gather(y_v, [s16])
            plsc.addupdate_scatter(yacc_v, [dl16], vals * w16)
            return 0

        lax.fori_loop(0, _CAP // 16, body, 0)
        pltpu.sync_copy(yacc_v, pyp_hbm.at[wid, c])


def _run_pb(y, bsrc, bdl, bew):
    f = pl.kernel(
        _pb_body,
        out_type=jax.ShapeDtypeStruct((_NW, _NCHUNK, _CHB), jnp.float32),
        mesh=_sc_mesh(),
        compiler_params=pltpu.CompilerParams(needs_layout_passes=False),
        scratch_types=[
            pltpu.VMEM((_N,), jnp.float32),
            pltpu.VMEM((_CAP,), jnp.int32),
            pltpu.VMEM((_CAP,), jnp.int32),
            pltpu.VMEM((_CAP,), jnp.float32),
            pltpu.VMEM((_CHB,), jnp.float32),
        ],
    )
    return f(y, bsrc, bdl, bew)


def _pc_body(g_hbm, bsrc2_hbm, bdl2_hbm, bew2_hbm, cnts2_hbm, out_hbm,
             sraw_v, draw_v, wraw_v, cslab_v, src_c, dl_c, ew_c,
             rows0_v, rows1_v, acc_v, sem0, sem1):
    core = lax.axis_index("c")
    sub = lax.axis_index("s")
    iota = lax.iota(jnp.int32, 16)
    zf = jnp.zeros((16,), jnp.float32)
    zi = jnp.zeros((16,), jnp.int32)
    cols = [iota + j * 16 for j in range(16)]

    def run_chunk(cc, _):
        chunk = core * 8 + cc

        def za(i, _2):
            acc_v[pl.ds(i * 16, 16)] = zf
            return 0

        lax.fori_loop(0, 256 * 256 // 16, za, 0)

        pltpu.sync_copy(bsrc2_hbm.at[chunk, sub], sraw_v)
        pltpu.sync_copy(bdl2_hbm.at[chunk, sub], draw_v)
        pltpu.sync_copy(bew2_hbm.at[chunk, sub], wraw_v)
        pltpu.sync_copy(cnts2_hbm.at[chunk], cslab_v)

        cur = jnp.int32(0)
        for w in range(_NW):
            crow = cslab_v[w, pl.ds(0, 16)]
            c_w = jnp.max(jnp.where(iota == sub, crow, 0))
            for j in range(_S2CAP // 16):
                m = (cols[j] if j < 16 else iota + j * 16) < c_w
                plsc.store_compressed(src_c.at[pl.ds(cur, 16)],
                                      sraw_v[w, pl.ds(j * 16, 16)], mask=m)
                plsc.store_compressed(dl_c.at[pl.ds(cur, 16)],
                                      draw_v[w, pl.ds(j * 16, 16)], mask=m)
                plsc.store_compressed(ew_c.at[pl.ds(cur, 16)],
                                      wraw_v[w, pl.ds(j * 16, 16)], mask=m)
                take = jnp.clip(c_w - j * 16, 0, 16)
                cur = cur + take
        ncomp = cur
        t0 = (ncomp // 16) * 16
        for k in range(5):
            pos = t0 + k * 16
            m = (iota + pos) >= ncomp
            sv = src_c[pl.ds(pos, 16)]
            src_c[pl.ds(pos, 16)] = jnp.where(m, 0, sv)
            wv = ew_c[pl.ds(pos, 16)]
            ew_c[pl.ds(pos, 16)] = jnp.where(m, 0.0, wv)
            dv = dl_c[pl.ds(pos, 16)]
            dl_c[pl.ds(pos, 16)] = jnp.where(m, 0, dv)
        nb = (ncomp + 63) // 64

        def issue(b, rows_v, sem):
            return pltpu.async_copy(
                g_hbm.at[src_c.at[pl.ds(b * 64, 64)]], rows_v, sem)

        def process(b, rows_v):
            def grp(g, _3):
                w16 = ew_c[pl.ds(b * 64 + g * 16, 16)]
                dl16 = lax.bitwise_and(dl_c[pl.ds(b * 64 + g * 16, 16)], 255)
                base16 = dl16 * 256
                for r in range(16):
                    e = g * 16 + r
                    w = w16[r]
                    bvec = iota * 0 + base16[r]
                    for j in range(16):
                        v = rows_v[e, pl.ds(j * 16, 16)] * w
                        plsc.addupdate_scatter(acc_v, [bvec + cols[j]], v)
                return 0

            lax.fori_loop(0, 4, grp, 0)

        @pl.when(nb > 0)
        def _prime():
            issue(0, rows0_v, sem0)

        def pair(p, _2):
            b0 = 2 * p
            b1 = b0 + 1
            pltpu.make_async_copy(
                g_hbm.at[src_c.at[pl.ds(b0 * 64, 64)]], rows0_v, sem0).wait()

            @pl.when(b1 < nb)
            def _i1():
                issue(b1, rows1_v, sem1)

            process(b0, rows0_v)

            @pl.when(b1 < nb)
            def _p1():
                pltpu.make_async_copy(
                    g_hbm.at[src_c.at[pl.ds(b1 * 64, 64)]], rows1_v,
                    sem1).wait()

                @pl.when(b1 + 1 < nb)
                def _i2():
                    issue(b1 + 1, rows0_v, sem0)

                process(b1, rows1_v)

            return 0

        lax.fori_loop(0, (nb + 1) // 2, pair, 0)

        off = chunk * _CHB + sub * 256

        @pl.when(off < _N)
        def _wb():
            pltpu.sync_copy(acc_v, out_hbm.at[pl.ds(off * 256, 256 * 256)])

        return 0

    lax.fori_loop(0, 8, run_chunk, 0)


def _run_pc(g, bsrc2, bdl2, bew2, cnts2):
    f = pl.kernel(
        _pc_body,
        out_type=jax.ShapeDtypeStruct((_N * 256,), jnp.float32),
        mesh=_sc_mesh(),
        compiler_params=pltpu.CompilerParams(needs_layout_passes=False),
        scratch_types=[
            pltpu.VMEM((_NW, _S2CAP), jnp.int32),
            pltpu.VMEM((_NW, _S2CAP), jnp.int32),
            pltpu.VMEM((_NW, _S2CAP), jnp.float32),
            pltpu.VMEM((_NW, 16), jnp.int32),
            pltpu.VMEM((_NW * _S2CAP + 80,), jnp.int32),
            pltpu.VMEM((_NW * _S2CAP + 80,), jnp.int32),
            pltpu.VMEM((_NW * _S2CAP + 80,), jnp.float32),
            pltpu.VMEM((64, 256), jnp.float32),
            pltpu.VMEM((64, 256), jnp.float32),
            pltpu.VMEM((256 * 256,), jnp.float32),
            pltpu.SemaphoreType.DMA,
            pltpu.SemaphoreType.DMA,
        ],
    )
    return f(g, bsrc2, bdl2, bew2, cnts2).reshape(_N, 256)


def _mm_body(h_ref, w_ref, al_ref, be_ref, dv_ref, b_ref, o_ref, acc_ref,
             *, act, affine, rowscale):
    k = pl.program_id(2)

    @pl.when(k == 0)
    def _init():
        acc_ref[...] = jnp.zeros_like(acc_ref)

    h = h_ref[...]
    if affine:
        h = h * al_ref[...] + be_ref[...]
    acc_ref[...] += jnp.dot(h, w_ref[...], preferred_element_type=jnp.float32)

    @pl.when(k == pl.num_programs(2) - 1)
    def _fin():
        r = acc_ref[...] + b_ref[...]
        if rowscale:
            r = r * dv_ref[...]
        if act:
            r = jnp.maximum(r, 0.0)
        o_ref[...] = r


def _matmul(h, W, b, bm, bn, bk, act=False, affine=None, dinv=None):
    M, K = h.shape
    K2, Nn = W.shape
    assert K == K2 and M % bm == 0 and Nn % bn == 0 and K % bk == 0
    if affine is None:
        al = jnp.ones((1, K), jnp.float32)
        bt = jnp.zeros((1, K), jnp.float32)
    else:
        al, bt = affine
        al = al.reshape(1, K)
        bt = bt.reshape(1, K)
    dv = jnp.ones((M, 1), jnp.float32) if dinv is None else dinv.reshape(M, 1)
    return pl.pallas_call(
        functools.partial(_mm_body, act=act, affine=affine is not None,
                          rowscale=dinv is not None),
        grid=(M // bm, Nn // bn, K // bk),
        in_specs=[
            pl.BlockSpec((bm, bk), lambda m, i, k: (m, k)),
            pl.BlockSpec((bk, bn), lambda m, i, k: (k, i)),
            pl.BlockSpec((1, bk), lambda m, i, k: (0, k)),
            pl.BlockSpec((1, bk), lambda m, i, k: (0, k)),
            pl.BlockSpec((bm, 1), lambda m, i, k: (m, 0)),
            pl.BlockSpec((1, bn), lambda m, i, k: (0, i)),
        ],
        out_specs=pl.BlockSpec((bm, bn), lambda m, i, k: (m, i)),
        out_shape=jax.ShapeDtypeStruct((M, Nn), jnp.float32),
        scratch_shapes=[pltpu.VMEM((bm, bn), jnp.float32)],
        compiler_params=pltpu.CompilerParams(
            dimension_semantics=("parallel", "parallel", "arbitrary"),
        ),
    )(h, W, al, bt, dv, b.reshape(1, -1))


def _stats_body(s_ref, g_ref, dv_ref, b_ref, r_ref, st_ref, sacc_ref):
    i = pl.program_id(0)

    @pl.when(i == 0)
    def _init():
        sacc_ref[...] = jnp.zeros_like(sacc_ref)

    r = jnp.maximum(dv_ref[...] * (s_ref[...] + g_ref[...]) + b_ref[...],
                    0.0)
    r_ref[...] = r
    sacc_ref[0, :] += jnp.sum(r, axis=0)
    sacc_ref[1, :] += jnp.sum(r * r, axis=0)

    @pl.when(i == pl.num_programs(0) - 1)
    def _fin():
        st_ref[...] = sacc_ref[...]


def _relu_stats(s, g, dinv, b, bm=3968):
    M = s.shape[0]
    return pl.pallas_call(
        _stats_body,
        grid=(M // bm,),
        in_specs=[
            pl.BlockSpec((bm, 256), lambda i: (i, 0)),
            pl.BlockSpec((bm, 256), lambda i: (i, 0)),
            pl.BlockSpec((bm, 1), lambda i: (i, 0)),
            pl.BlockSpec((1, 256), lambda i: (0, 0)),
        ],
        out_specs=[
            pl.BlockSpec((bm, 256), lambda i: (i, 0)),
            pl.BlockSpec((2, 256), lambda i: (0, 0)),
        ],
        out_shape=[
            jax.ShapeDtypeStruct((M, 256), jnp.float32),
            jax.ShapeDtypeStruct((2, 256), jnp.float32),
        ],
        scratch_shapes=[pltpu.VMEM((2, 256), jnp.float32)],
        compiler_params=pltpu.CompilerParams(
            dimension_semantics=("arbitrary",),
        ),
    )(s, g, dinv.reshape(M, 1), b.reshape(1, 256))


def _l1_body(ax_ref, w_ref, b_ref, r_ref, st_ref, sacc_ref):
    i = pl.program_id(0)

    @pl.when(i == 0)
    def _init():
        sacc_ref[...] = jnp.zeros_like(sacc_ref)

    r = jnp.maximum(ax_ref[...] * w_ref[...] + b_ref[...], 0.0)
    r_ref[...] = r
    sacc_ref[0, :] += jnp.sum(r, axis=0)
    sacc_ref[1, :] += jnp.sum(r * r, axis=0)

    @pl.when(i == pl.num_programs(0) - 1)
    def _fin():
        st_ref[...] = sacc_ref[...]


def _l1_relu_stats(ax, w0, b0, bm=3968):
    M = ax.shape[0]
    return pl.pallas_call(
        _l1_body,
        grid=(M // bm,),
        in_specs=[
            pl.BlockSpec((bm, 1), lambda i: (i, 0)),
            pl.BlockSpec((1, 256), lambda i: (0, 0)),
            pl.BlockSpec((1, 256), lambda i: (0, 0)),
        ],
        out_specs=[
            pl.BlockSpec((bm, 256), lambda i: (i, 0)),
            pl.BlockSpec((2, 256), lambda i: (0, 0)),
        ],
        out_shape=[
            jax.ShapeDtypeStruct((M, 256), jnp.float32),
            jax.ShapeDtypeStruct((2, 256), jnp.float32),
        ],
        scratch_shapes=[pltpu.VMEM((2, 256), jnp.float32)],
        compiler_params=pltpu.CompilerParams(
            dimension_semantics=("arbitrary",),
        ),
    )(ax.reshape(M, 1), w0.reshape(1, 256), b0.reshape(1, 256))


def _affine_from_stats(st, g, be):
    mu = st[0] / _N
    var = st[1] / _N - mu * mu
    al = g * lax.rsqrt(var + _EPS)
    return al, be - mu * al


def kernel(x, edge_index, edge_weights, W0, b0, g0, be0, W1, b1, g1, be1,
           W2, b2, g2, be2, Wf1, bf1, Wf2, bf2, Wo, bo):
    src = edge_index[0]
    dst = edge_index[1]

    (bsrc, bdl, bew, cnts, degp,
     bsrc2, bdl2, bew2, cnts2) = _run_pa(src, dst, edge_weights)
    bsrc3 = bsrc.reshape(_NW, _NCHUNK, _CAP)
    bdl3 = bdl.reshape(_NW, _NCHUNK, _CAP)
    bew3 = bew.reshape(_NW, _NCHUNK, _CAP)

    deg = degp.reshape(_NW, _NCHUNK * _CHB).sum(axis=0)[:_N] + 1.0
    dinv = lax.rsqrt(deg)
    y = dinv * x[:, 0]

    pyp = _run_pb(y, bsrc3, bdl3, bew3)
    sy = pyp.reshape(_NW, _NCHUNK * _CHB).sum(axis=0)[:_N]
    ax = dinv * (sy + y)

    r1, st1 = _l1_relu_stats(ax, W0[0], b0)
    al1, bt1 = _affine_from_stats(st1, g0, be0)

    G2 = _matmul(r1, W1, jnp.zeros((256,), jnp.float32), bm=3968, bn=256,
                 bk=256, affine=(al1, bt1), dinv=dinv)
    s2 = _run_pc(G2, bsrc2, bdl2, bew2, cnts2)
    r2, st2 = _relu_stats(s2, G2, dinv, b1)
    al2, bt2 = _affine_from_stats(st2, g1, be1)

    G3 = _matmul(r2, W2, jnp.zeros((256,), jnp.float32), bm=3968, bn=256,
                 bk=256, affine=(al2, bt2), dinv=dinv)
    s3 = _run_pc(G3, bsrc2, bdl2, bew2, cnts2)
    r3, st3 = _relu_stats(s3, G3, dinv, b2)
    al3, bt3 = _affine_from_stats(st3, g2, be2)

    h = r3.reshape(_B, _NPB * 256)
    alf = jnp.tile(al3, _NPB)
    btf = jnp.tile(bt3, _NPB)
    h = _matmul(h, Wf1, bf1, bm=_B, bn=1024, bk=512, act=True,
                affine=(alf, btf))
    h = _matmul(h, Wf2, bf2, bm=_B, bn=1024, bk=512, act=True)
    return _matmul(h, Wo, bo, bm=_B, bn=256, bk=512)


# speedup vs baseline: 1.0604x; 1.0604x over previous
import functools

import jax
import jax.numpy as jnp
from jax import lax
from jax.experimental import pallas as pl
from jax.experimental.pallas import tpu as pltpu
from jax.experimental.pallas import tpu_sc as plsc

_B = 1024
_NPB = 62
_N = _B * _NPB
_E = 317440
_NW = 32
_EPW = _E // _NW
_EH = _EPW // 2
_NCHUNK = 16
_CHB = 4096
_CAP = 2048
_S2CAP = 128
_EPS = 1e-5


def _sc_mesh():
    return plsc.VectorSubcoreMesh(core_axis_name="c", subcore_axis_name="s",
                                  num_cores=2, num_subcores=16)


def _wid():
    return lax.axis_index("s") * 2 + lax.axis_index("c")


def _zero_ref(ref, n, dtype):
    z = jnp.zeros((16,), dtype)

    def body(i, _):
        ref[pl.ds(i * 16, 16)] = z
        return 0

    lax.fori_loop(0, n // 16, body, 0)


def _pa_body(src_hbm, dst_hbm, ew_hbm,
             bsrc_hbm, bdl_hbm, bew_hbm, cnts_hbm, degp_hbm,
             bsrc2_hbm, bdl2_hbm, bew2_hbm, cnts2_hbm,
             es_v, ed_v, ee_v, bsrc_v, bdl_v, bew_v, dacc_v, cnt_v,
             ss_v, sd_v, sw_v, cnt2_v):
    wid = _wid()
    iota = lax.iota(jnp.int32, 16)
    zi = jnp.zeros((16,), jnp.int32)
    zf = jnp.zeros((16,), jnp.float32)

    def zb(i, _):
        bsrc_v[pl.ds(i * 16, 16)] = zi
        bdl_v[pl.ds(i * 16, 16)] = zi
        bew_v[pl.ds(i * 16, 16)] = zf
        return 0

    lax.fori_loop(0, _NCHUNK * _CAP // 16, zb, 0)

    base = wid * _EPW
    curs = tuple(jnp.int32(0) for _ in range(_NCHUNK))
    for h in range(2):
        pltpu.sync_copy(src_hbm.at[pl.ds(base + h * _EH, _EH)], es_v)
        pltpu.sync_copy(dst_hbm.at[pl.ds(base + h * _EH, _EH)], ed_v)
        pltpu.sync_copy(ew_hbm.at[pl.ds(base + h * _EH, _EH)], ee_v)

        def vbody(i, cs):
            s16 = es_v[pl.ds(i * 16, 16)]
            d16 = ed_v[pl.ds(i * 16, 16)]
            w16 = ee_v[pl.ds(i * 16, 16)]
            ch = lax.shift_right_logical(d16, 12)
            dl = lax.bitwise_and(d16, 4095)
            new = []
            for c in range(_NCHUNK):
                m = ch == c
                pc = jnp.sum(m.astype(jnp.int32))
                cur = jnp.minimum(cs[c], _CAP - 16)
                off = cur + c * _CAP
                plsc.store_compressed(bsrc_v.at[pl.ds(off, 16)], s16,
                                      mask=m)
                plsc.store_compressed(bdl_v.at[pl.ds(off, 16)], dl,
                                      mask=m)
                plsc.store_compressed(bew_v.at[pl.ds(off, 16)], w16,
                                      mask=m)
                new.append(cur + pc)
            return tuple(new)

        curs = lax.fori_loop(0, _EH // 16, vbody, curs)

    cvec = jnp.zeros((16,), jnp.int32)
    for c in range(_NCHUNK):
        cvec = cvec + jnp.where(iota == c, curs[c], 0)
    cnt_v[...] = cvec
    pltpu.sync_copy(cnt_v, cnts_hbm.at[wid])
    pltpu.sync_copy(bsrc_v, bsrc_hbm.at[wid])
    pltpu.sync_copy(bdl_v, bdl_hbm.at[wid])
    pltpu.sync_copy(bew_v, bew_hbm.at[wid])

    for c in range(_NCHUNK):
        _zero_ref(dacc_v, _CHB, jnp.float32)

        def dbody(i, _):
            dl16 = bdl_v[pl.ds(c * _CAP + i * 16, 16)]
            w16 = bew_v[pl.ds(c * _CAP + i * 16, 16)]
            plsc.addupdate_scatter(dacc_v, [dl16], w16)
            return 0

        lax.fori_loop(0, _CAP // 16, dbody, 0)
        pltpu.sync_copy(dacc_v, degp_hbm.at[wid, c])

    zi16 = jnp.zeros((16,), jnp.int32)
    zf16 = jnp.zeros((16,), jnp.float32)

    def s2_chunk(c, _):
        def z2(i, _2):
            ss_v[pl.ds(i * 16, 16)] = zi16
            sd_v[pl.ds(i * 16, 16)] = zi16
            sw_v[pl.ds(i * 16, 16)] = zf16
            return 0

        lax.fori_loop(0, _S2CAP * 16 // 16, z2, 0)
        cnt_c = jnp.max(jnp.where(iota == c, cvec, 0))
        nv = (cnt_c + 15) // 16

        def s2v(i, cs2):
            base_e = c * _CAP + i * 16
            s16 = bsrc_v[pl.ds(base_e, 16)]
            d16 = bdl_v[pl.ds(base_e, 16)]
            w16 = bew_v[pl.ds(base_e, 16)]
            valid = (iota + i * 16) < cnt_c
            sb = lax.shift_right_logical(d16, 8)
            new = []
            for s in range(16):
                m = (sb == s) & valid
                pc = jnp.sum(m.astype(jnp.int32))
                cur = jnp.minimum(cs2[s], _S2CAP - 16)
                off = s * _S2CAP + cur
                plsc.store_compressed(ss_v.at[pl.ds(off, 16)], s16, mask=m)
                plsc.store_compressed(sd_v.at[pl.ds(off, 16)], d16, mask=m)
                plsc.store_compressed(sw_v.at[pl.ds(off, 16)], w16, mask=m)
                new.append(cur + pc)
            return tuple(new)

        cs2 = lax.fori_loop(0, nv, s2v,
                            tuple(jnp.int32(0) for _ in range(16)))
        c2v = jnp.zeros((16,), jnp.int32)
        for s in range(16):
            c2v = c2v + jnp.where(iota == s, cs2[s], 0)
        cnt2_v[...] = c2v
        pltpu.sync_copy(cnt2_v, cnts2_hbm.at[c, wid])
        for s in range(16):
            pltpu.sync_copy(ss_v.at[pl.ds(s * _S2CAP, _S2CAP)],
                            bsrc2_hbm.at[c, s, wid])
            pltpu.sync_copy(sd_v.at[pl.ds(s * _S2CAP, _S2CAP)],
                            bdl2_hbm.at[c, s, wid])
            pltpu.sync_copy(sw_v.at[pl.ds(s * _S2CAP, _S2CAP)],
                            bew2_hbm.at[c, s, wid])
        return 0

    lax.fori_loop(0, _NCHUNK, s2_chunk, 0)


def _run_pa(src, dst, ew):
    f = pl.kernel(
        _pa_body,
        out_type=[
            jax.ShapeDtypeStruct((_NW, _NCHUNK * _CAP), jnp.int32),
            jax.ShapeDtypeStruct((_NW, _NCHUNK * _CAP), jnp.int32),
            jax.ShapeDtypeStruct((_NW, _NCHUNK * _CAP), jnp.float32),
            jax.ShapeDtypeStruct((_NW, 16), jnp.int32),
            jax.ShapeDtypeStruct((_NW, _NCHUNK, _CHB), jnp.float32),
            jax.ShapeDtypeStruct((_NCHUNK, 16, _NW, _S2CAP), jnp.int32),
            jax.ShapeDtypeStruct((_NCHUNK, 16, _NW, _S2CAP), jnp.int32),
            jax.ShapeDtypeStruct((_NCHUNK, 16, _NW, _S2CAP), jnp.float32),
            jax.ShapeDtypeStruct((_NCHUNK, _NW, 16), jnp.int32),
        ],
        mesh=_sc_mesh(),
        compiler_params=pltpu.CompilerParams(needs_layout_passes=False),
        scratch_types=[
            pltpu.VMEM((_EH,), jnp.int32),
            pltpu.VMEM((_EH,), jnp.int32),
            pltpu.VMEM((_EH,), jnp.float32),
            pltpu.VMEM((_NCHUNK * _CAP,), jnp.int32),
            pltpu.VMEM((_NCHUNK * _CAP,), jnp.int32),
            pltpu.VMEM((_NCHUNK * _CAP,), jnp.float32),
            pltpu.VMEM((_CHB,), jnp.float32),
            pltpu.VMEM((16,), jnp.int32),
            pltpu.VMEM((_S2CAP * 16,), jnp.int32),
            pltpu.VMEM((_S2CAP * 16,), jnp.int32),
            pltpu.VMEM((_S2CAP * 16,), jnp.float32),
            pltpu.VMEM((16,), jnp.int32),
        ],
    )
    return f(src, dst, ew)


def _pb_body(y_hbm, bsrc_hbm, bdl_hbm, bew_hbm, pyp_hbm,
             y_v, src_v, dl_v, ew_v, yacc_v):
    wid = _wid()
    pltpu.sync_copy(y_hbm, y_v)
    for c in range(_NCHUNK):
        _zero_ref(yacc_v, _CHB, jnp.float32)
        pltpu.sync_copy(bsrc_hbm.at[wid, c], src_v)
        pltpu.sync_copy(bdl_hbm.at[wid, c], dl_v)
        pltpu.sync_copy(bew_hbm.at[wid, c], ew_v)

        def body(i, _):
            s16 = src_v[pl.ds(i * 16, 16)]
            dl16 = dl_v[pl.ds(i * 16, 16)]
            w16 = ew_v[pl.ds(i * 16, 16)]
            vals = plsc.load_gather(y_v, [s16])
            plsc.addupdate_scatter(yacc_v, [dl16], vals * w16)
            return 0

        lax.fori_loop(0, _CAP // 16, body, 0)
        pltpu.sync_copy(yacc_v, pyp_hbm.at[wid, c])


def _run_pb(y, bsrc, bdl, bew):
    f = pl.kernel(
        _pb_body,
        out_type=jax.ShapeDtypeStruct((_NW, _NCHUNK, _CHB), jnp.float32),
        mesh=_sc_mesh(),
        compiler_params=pltpu.CompilerParams(needs_layout_passes=False),
        scratch_types=[
            pltpu.VMEM((_N,), jnp.float32),
            pltpu.VMEM((_CAP,), jnp.int32),
            pltpu.VMEM((_CAP,), jnp.int32),
            pltpu.VMEM((_CAP,), jnp.float32),
            pltpu.VMEM((_CHB,), jnp.float32),
        ],
    )
    return f(y, bsrc, bdl, bew)


def _pc_body(g_hbm, bsrc2_hbm, bdl2_hbm, bew2_hbm, cnts2_hbm, out_hbm,
             sraw_v, draw_v, wraw_v, cslab_v, src_c, dl_c, ew_c,
             rows0_v, rows1_v, acc_v, sem0, sem1):
    core = lax.axis_index("c")
    sub = lax.axis_index("s")
    iota = lax.iota(jnp.int32, 16)
    zf = jnp.zeros((16,), jnp.float32)
    zi = jnp.zeros((16,), jnp.int32)
    cols = [iota + j * 16 for j in range(16)]

    def run_chunk(cc, _):
        chunk = core * 8 + cc

        def za(i, _2):
            for j in range(16):
                acc_v[pl.ds(i * 256 + j * 16, 16)] = zf
            return 0

        lax.fori_loop(0, 256, za, 0)

        pltpu.sync_copy(bsrc2_hbm.at[chunk, sub], sraw_v)
        pltpu.sync_copy(bdl2_hbm.at[chunk, sub], draw_v)
        pltpu.sync_copy(bew2_hbm.at[chunk, sub], wraw_v)
        pltpu.sync_copy(cnts2_hbm.at[chunk], cslab_v)

        cur = jnp.int32(0)
        for w in range(_NW):
            crow = cslab_v[w, pl.ds(0, 16)]
            c_w = jnp.max(jnp.where(iota == sub, crow, 0))
            for j in range(_S2CAP // 16):
                m = (cols[j] if j < 16 else iota + j * 16) < c_w
                plsc.store_compressed(src_c.at[pl.ds(cur, 16)],
                                      sraw_v[w, pl.ds(j * 16, 16)], mask=m)
                plsc.store_compressed(dl_c.at[pl.ds(cur, 16)],
                                      draw_v[w, pl.ds(j * 16, 16)], mask=m)
                plsc.store_compressed(ew_c.at[pl.ds(cur, 16)],
                                      wraw_v[w, pl.ds(j * 16, 16)], mask=m)
                take = jnp.clip(c_w - j * 16, 0, 16)
                cur = cur + take
        ncomp = cur
        t0 = (ncomp // 16) * 16
        for k in range(5):
            pos = t0 + k * 16
            m = (iota + pos) >= ncomp
            sv = src_c[pl.ds(pos, 16)]
            src_c[pl.ds(pos, 16)] = jnp.where(m, 0, sv)
            wv = ew_c[pl.ds(pos, 16)]
            ew_c[pl.ds(pos, 16)] = jnp.where(m, 0.0, wv)
            dv = dl_c[pl.ds(pos, 16)]
            dl_c[pl.ds(pos, 16)] = jnp.where(m, 0, dv)
        nb = (ncomp + 63) // 64

        def issue(b, rows_v, sem):
            return pltpu.async_copy(
                g_hbm.at[src_c.at[pl.ds(b * 64, 64)]], rows_v, sem)

        def process(b, rows_v):
            def grp(g, _3):
                w16 = ew_c[pl.ds(b * 64 + g * 16, 16)]
                dl16 = lax.bitwise_and(dl_c[pl.ds(b * 64 + g * 16, 16)], 255)
                base16 = dl16 * 256
                for r in range(16):
                    e = g * 16 + r
                    w = w16[r]
                    bvec = iota * 0 + base16[r]
                    for j in range(16):
                        v = rows_v[e, pl.ds(j * 16, 16)] * w
                        plsc.addupdate_scatter(acc_v, [bvec + cols[j]], v)
                return 0

            lax.fori_loop(0, 4, grp, 0)

        @pl.when(nb > 0)
        def _prime():
            issue(0, rows0_v, sem0)

        def pair(p, _2):
            b0 = 2 * p
            b1 = b0 + 1
            pltpu.make_async_copy(
                g_hbm.at[src_c.at[pl.ds(b0 * 64, 64)]], rows0_v, sem0).wait()

            @pl.when(b1 < nb)
            def _i1():
                issue(b1, rows1_v, sem1)

            process(b0, rows0_v)

            @pl.when(b1 < nb)
            def _p1():
                pltpu.make_async_copy(
                    g_hbm.at[src_c.at[pl.ds(b1 * 64, 64)]], rows1_v,
                    sem1).wait()

                @pl.when(b1 + 1 < nb)
                def _i2():
                    issue(b1 + 1, rows0_v, sem0)

                process(b1, rows1_v)

            return 0

        lax.fori_loop(0, (nb + 1) // 2, pair, 0)

        off = chunk * _CHB + sub * 256

        @pl.when(off < _N)
        def _wb():
            pltpu.sync_copy(acc_v, out_hbm.at[pl.ds(off * 256, 256 * 256)])

        return 0

    lax.fori_loop(0, 8, run_chunk, 0)


def _run_pc(g, bsrc2, bdl2, bew2, cnts2):
    f = pl.kernel(
        _pc_body,
        out_type=jax.ShapeDtypeStruct((_N * 256,), jnp.float32),
        mesh=_sc_mesh(),
        compiler_params=pltpu.CompilerParams(needs_layout_passes=False),
        scratch_types=[
            pltpu.VMEM((_NW, _S2CAP), jnp.int32),
            pltpu.VMEM((_NW, _S2CAP), jnp.int32),
            pltpu.VMEM((_NW, _S2CAP), jnp.float32),
            pltpu.VMEM((_NW, 16), jnp.int32),
            pltpu.VMEM((_NW * _S2CAP + 80,), jnp.int32),
            pltpu.VMEM((_NW * _S2CAP + 80,), jnp.int32),
            pltpu.VMEM((_NW * _S2CAP + 80,), jnp.float32),
            pltpu.VMEM((64, 256), jnp.float32),
            pltpu.VMEM((64, 256), jnp.float32),
            pltpu.VMEM((256 * 256,), jnp.float32),
            pltpu.SemaphoreType.DMA,
            pltpu.SemaphoreType.DMA,
        ],
    )
    return f(g, bsrc2, bdl2, bew2, cnts2).reshape(_N, 256)


def _mm_body(h_ref, w_ref, al_ref, be_ref, dv_ref, b_ref, o_ref, acc_ref,
             *, act, affine, rowscale):
    k = pl.program_id(2)

    @pl.when(k == 0)
    def _init():
        acc_ref[...] = jnp.zeros_like(acc_ref)

    h = h_ref[...]
    if affine:
        h = h * al_ref[...] + be_ref[...]
    acc_ref[...] += jnp.dot(h, w_ref[...], preferred_element_type=jnp.float32)

    @pl.when(k == pl.num_programs(2) - 1)
    def _fin():
        r = acc_ref[...] + b_ref[...]
        if rowscale:
            r = r * dv_ref[...]
        if act:
            r = jnp.maximum(r, 0.0)
        o_ref[...] = r


def _matmul(h, W, b, bm, bn, bk, act=False, affine=None, dinv=None):
    M, K = h.shape
    K2, Nn = W.shape
    assert K == K2 and M % bm == 0 and Nn % bn == 0 and K % bk == 0
    if affine is None:
        al = jnp.ones((1, K), jnp.float32)
        bt = jnp.zeros((1, K), jnp.float32)
    else:
        al, bt = affine
        al = al.reshape(1, K)
        bt = bt.reshape(1, K)
    dv = jnp.ones((M, 1), jnp.float32) if dinv is None else dinv.reshape(M, 1)
    return pl.pallas_call(
        functools.partial(_mm_body, act=act, affine=affine is not None,
                          rowscale=dinv is not None),
        grid=(M // bm, Nn // bn, K // bk),
        in_specs=[
            pl.BlockSpec((bm, bk), lambda m, i, k: (m, k)),
            pl.BlockSpec((bk, bn), lambda m, i, k: (k, i)),
            pl.BlockSpec((1, bk), lambda m, i, k: (0, k)),
            pl.BlockSpec((1, bk), lambda m, i, k: (0, k)),
            pl.BlockSpec((bm, 1), lambda m, i, k: (m, 0)),
            pl.BlockSpec((1, bn), lambda m, i, k: (0, i)),
        ],
        out_specs=pl.BlockSpec((bm, bn), lambda m, i, k: (m, i)),
        out_shape=jax.ShapeDtypeStruct((M, Nn), jnp.float32),
        scratch_shapes=[pltpu.VMEM((bm, bn), jnp.float32)],
        compiler_params=pltpu.CompilerParams(
            dimension_semantics=("parallel", "parallel", "arbitrary"),
        ),
    )(h, W, al, bt, dv, b.reshape(1, -1))


def _stats_body(s_ref, g_ref, dv_ref, b_ref, r_ref, st_ref, sacc_ref):
    i = pl.program_id(0)

    @pl.when(i == 0)
    def _init():
        sacc_ref[...] = jnp.zeros_like(sacc_ref)

    r = jnp.maximum(dv_ref[...] * (s_ref[...] + g_ref[...]) + b_ref[...],
                    0.0)
    r_ref[...] = r
    sacc_ref[0, :] += jnp.sum(r, axis=0)
    sacc_ref[1, :] += jnp.sum(r * r, axis=0)

    @pl.when(i == pl.num_programs(0) - 1)
    def _fin():
        st_ref[...] = sacc_ref[...]


def _relu_stats(s, g, dinv, b, bm=3968):
    M = s.shape[0]
    return pl.pallas_call(
        _stats_body,
        grid=(M // bm,),
        in_specs=[
            pl.BlockSpec((bm, 256), lambda i: (i, 0)),
            pl.BlockSpec((bm, 256), lambda i: (i, 0)),
            pl.BlockSpec((bm, 1), lambda i: (i, 0)),
            pl.BlockSpec((1, 256), lambda i: (0, 0)),
        ],
        out_specs=[
            pl.BlockSpec((bm, 256), lambda i: (i, 0)),
            pl.BlockSpec((2, 256), lambda i: (0, 0)),
        ],
        out_shape=[
            jax.ShapeDtypeStruct((M, 256), jnp.float32),
            jax.ShapeDtypeStruct((2, 256), jnp.float32),
        ],
        scratch_shapes=[pltpu.VMEM((2, 256), jnp.float32)],
        compiler_params=pltpu.CompilerParams(
            dimension_semantics=("arbitrary",),
        ),
    )(s, g, dinv.reshape(M, 1), b.reshape(1, 256))


def _l1_body(ax_ref, w_ref, b_ref, r_ref, st_ref, sacc_ref):
    i = pl.program_id(0)

    @pl.when(i == 0)
    def _init():
        sacc_ref[...] = jnp.zeros_like(sacc_ref)

    r = jnp.maximum(ax_ref[...] * w_ref[...] + b_ref[...], 0.0)
    r_ref[...] = r
    sacc_ref[0, :] += jnp.sum(r, axis=0)
    sacc_ref[1, :] += jnp.sum(r * r, axis=0)

    @pl.when(i == pl.num_programs(0) - 1)
    def _fin():
        st_ref[...] = sacc_ref[...]


def _l1_relu_stats(ax, w0, b0, bm=3968):
    M = ax.shape[0]
    return pl.pallas_call(
        _l1_body,
        grid=(M // bm,),
        in_specs=[
            pl.BlockSpec((bm, 1), lambda i: (i, 0)),
            pl.BlockSpec((1, 256), lambda i: (0, 0)),
            pl.BlockSpec((1, 256), lambda i: (0, 0)),
        ],
        out_specs=[
            pl.BlockSpec((bm, 256), lambda i: (i, 0)),
            pl.BlockSpec((2, 256), lambda i: (0, 0)),
        ],
        out_shape=[
            jax.ShapeDtypeStruct((M, 256), jnp.float32),
            jax.ShapeDtypeStruct((2, 256), jnp.float32),
        ],
        scratch_shapes=[pltpu.VMEM((2, 256), jnp.float32)],
        compiler_params=pltpu.CompilerParams(
            dimension_semantics=("arbitrary",),
        ),
    )(ax.reshape(M, 1), w0.reshape(1, 256), b0.reshape(1, 256))


def _affine_from_stats(st, g, be):
    mu = st[0] / _N
    var = st[1] / _N - mu * mu
    al = g * lax.rsqrt(var + _EPS)
    return al, be - mu * al


def kernel(x, edge_index, edge_weights, W0, b0, g0, be0, W1, b1, g1, be1,
           W2, b2, g2, be2, Wf1, bf1, Wf2, bf2, Wo, bo):
    src = edge_index[0]
    dst = edge_index[1]

    (bsrc, bdl, bew, cnts, degp,
     bsrc2, bdl2, bew2, cnts2) = _run_pa(src, dst, edge_weights)
    bsrc3 = bsrc.reshape(_NW, _NCHUNK, _CAP)
    bdl3 = bdl.reshape(_NW, _NCHUNK, _CAP)
    bew3 = bew.reshape(_NW, _NCHUNK, _CAP)

    deg = degp.reshape(_NW, _NCHUNK * _CHB).sum(axis=0)[:_N] + 1.0
    dinv = lax.rsqrt(deg)
    y = dinv * x[:, 0]

    pyp = _run_pb(y, bsrc3, bdl3, bew3)
    sy = pyp.reshape(_NW, _NCHUNK * _CHB).sum(axis=0)[:_N]
    ax = dinv * (sy + y)

    r1, st1 = _l1_relu_stats(ax, W0[0], b0)
    al1, bt1 = _affine_from_stats(st1, g0, be0)

    G2 = _matmul(r1, W1, jnp.zeros((256,), jnp.float32), bm=3968, bn=256,
                 bk=256, affine=(al1, bt1), dinv=dinv)
    s2 = _run_pc(G2, bsrc2, bdl2, bew2, cnts2)
    r2, st2 = _relu_stats(s2, G2, dinv, b1)
    al2, bt2 = _affine_from_stats(st2, g1, be1)

    G3 = _matmul(r2, W2, jnp.zeros((256,), jnp.float32), bm=3968, bn=256,
                 bk=256, affine=(al2, bt2), dinv=dinv)
    s3 = _run_pc(G3, bsrc2, bdl2, bew2, cnts2)
    r3, st3 = _relu_stats(s3, G3, dinv, b2)
    al3, bt3 = _affine_from_stats(st3, g2, be2)

    h = r3.reshape(_B, _NPB * 256)
    alf = jnp.tile(al3, _NPB)
    btf = jnp.tile(bt3, _NPB)
    h = _matmul(h, Wf1, bf1, bm=_B, bn=1024, bk=512, act=True,
                affine=(alf, btf))
    h = _matmul(h, Wf2, bf2, bm=_B, bn=1024, bk=512, act=True)
    return _matmul(h, Wo, bo, bm=_B, bn=256, bk=512)


# speedup vs baseline: 1.1103x; 1.0471x over previous
import functools

import jax
import jax.numpy as jnp
from jax import lax
from jax.experimental import pallas as pl
from jax.experimental.pallas import tpu as pltpu
from jax.experimental.pallas import tpu_sc as plsc

_B = 1024
_NPB = 62
_N = _B * _NPB
_E = 317440
_NW = 32
_EPW = _E // _NW
_EH = _EPW // 2
_NCHUNK = 16
_CHB = 4096
_CAP = 2048
_S2CAP = 128
_EPS = 1e-5


def _sc_mesh():
    return plsc.VectorSubcoreMesh(core_axis_name="c", subcore_axis_name="s",
                                  num_cores=2, num_subcores=16)


def _wid():
    return lax.axis_index("s") * 2 + lax.axis_index("c")


def _zero_ref(ref, n, dtype):
    z = jnp.zeros((16,), dtype)

    def body(i, _):
        ref[pl.ds(i * 16, 16)] = z
        return 0

    lax.fori_loop(0, n // 16, body, 0)


def _pa_body(src_hbm, dst_hbm, ew_hbm,
             bsrc_hbm, bdl_hbm, bew_hbm, cnts_hbm, degp_hbm,
             bsrc2_hbm, bdl2_hbm, bew2_hbm, cnts2_hbm,
             es_v, ed_v, ee_v, bsrc_v, bdl_v, bew_v, dacc_v, cnt_v,
             ss_v, sd_v, sw_v, cnt2_v):
    wid = _wid()
    iota = lax.iota(jnp.int32, 16)
    zi = jnp.zeros((16,), jnp.int32)
    zf = jnp.zeros((16,), jnp.float32)

    def zb(i, _):
        bsrc_v[pl.ds(i * 16, 16)] = zi
        bdl_v[pl.ds(i * 16, 16)] = zi
        bew_v[pl.ds(i * 16, 16)] = zf
        return 0

    lax.fori_loop(0, _NCHUNK * _CAP // 16, zb, 0)

    base = wid * _EPW
    curs = tuple(jnp.int32(0) for _ in range(_NCHUNK))
    for h in range(2):
        pltpu.sync_copy(src_hbm.at[pl.ds(base + h * _EH, _EH)], es_v)
        pltpu.sync_copy(dst_hbm.at[pl.ds(base + h * _EH, _EH)], ed_v)
        pltpu.sync_copy(ew_hbm.at[pl.ds(base + h * _EH, _EH)], ee_v)

        def vbody(i, cs):
            s16 = es_v[pl.ds(i * 16, 16)]
            d16 = ed_v[pl.ds(i * 16, 16)]
            w16 = ee_v[pl.ds(i * 16, 16)]
            ch = lax.shift_right_logical(d16, 12)
            dl = lax.bitwise_and(d16, 4095)
            new = []
            for c in range(_NCHUNK):
                m = ch == c
                pc = jnp.sum(m.astype(jnp.int32))
                cur = jnp.minimum(cs[c], _CAP - 16)
                off = cur + c * _CAP
                plsc.store_compressed(bsrc_v.at[pl.ds(off, 16)], s16,
                                      mask=m)
                plsc.store_compressed(bdl_v.at[pl.ds(off, 16)], dl,
                                      mask=m)
                plsc.store_compressed(bew_v.at[pl.ds(off, 16)], w16,
                                      mask=m)
                new.append(cur + pc)
            return tuple(new)

        curs = lax.fori_loop(0, _EH // 16, vbody, curs)

    cvec = jnp.zeros((16,), jnp.int32)
    for c in range(_NCHUNK):
        cvec = cvec + jnp.where(iota == c, curs[c], 0)
    cnt_v[...] = cvec
    pltpu.sync_copy(cnt_v, cnts_hbm.at[wid])
    pltpu.sync_copy(bsrc_v, bsrc_hbm.at[wid])
    pltpu.sync_copy(bdl_v, bdl_hbm.at[wid])
    pltpu.sync_copy(bew_v, bew_hbm.at[wid])

    for c in range(_NCHUNK):
        _zero_ref(dacc_v, _CHB, jnp.float32)

        def dbody(i, _):
            dl16 = bdl_v[pl.ds(c * _CAP + i * 16, 16)]
            w16 = bew_v[pl.ds(c * _CAP + i * 16, 16)]
            plsc.addupdate_scatter(dacc_v, [dl16], w16)
            return 0

        lax.fori_loop(0, _CAP // 16, dbody, 0)
        pltpu.sync_copy(dacc_v, degp_hbm.at[wid, c])

    zi16 = jnp.zeros((16,), jnp.int32)
    zf16 = jnp.zeros((16,), jnp.float32)

    def s2_chunk(c, _):
        def z2(i, _2):
            ss_v[pl.ds(i * 16, 16)] = zi16
            sd_v[pl.ds(i * 16, 16)] = zi16
            sw_v[pl.ds(i * 16, 16)] = zf16
            return 0

        lax.fori_loop(0, _S2CAP * 16 // 16, z2, 0)
        cnt_c = jnp.max(jnp.where(iota == c, cvec, 0))
        nv = (cnt_c + 15) // 16

        def s2v(i, cs2):
            base_e = c * _CAP + i * 16
            s16 = bsrc_v[pl.ds(base_e, 16)]
            d16 = bdl_v[pl.ds(base_e, 16)]
            w16 = bew_v[pl.ds(base_e, 16)]
            valid = (iota + i * 16) < cnt_c
            sb = lax.shift_right_logical(d16, 8)
            new = []
            for s in range(16):
                m = (sb == s) & valid
                pc = jnp.sum(m.astype(jnp.int32))
                cur = jnp.minimum(cs2[s], _S2CAP - 16)
                off = s * _S2CAP + cur
                plsc.store_compressed(ss_v.at[pl.ds(off, 16)], s16, mask=m)
                plsc.store_compressed(sd_v.at[pl.ds(off, 16)], d16, mask=m)
                plsc.store_compressed(sw_v.at[pl.ds(off, 16)], w16, mask=m)
                new.append(cur + pc)
            return tuple(new)

        cs2 = lax.fori_loop(0, nv, s2v,
                            tuple(jnp.int32(0) for _ in range(16)))
        c2v = jnp.zeros((16,), jnp.int32)
        for s in range(16):
            c2v = c2v + jnp.where(iota == s, cs2[s], 0)
        cnt2_v[...] = c2v
        pltpu.sync_copy(cnt2_v, cnts2_hbm.at[c, wid])
        for s in range(16):
            pltpu.sync_copy(ss_v.at[pl.ds(s * _S2CAP, _S2CAP)],
                            bsrc2_hbm.at[c, s, wid])
            pltpu.sync_copy(sd_v.at[pl.ds(s * _S2CAP, _S2CAP)],
                            bdl2_hbm.at[c, s, wid])
            pltpu.sync_copy(sw_v.at[pl.ds(s * _S2CAP, _S2CAP)],
                            bew2_hbm.at[c, s, wid])
        return 0

    lax.fori_loop(0, _NCHUNK, s2_chunk, 0)


def _run_pa(src, dst, ew):
    f = pl.kernel(
        _pa_body,
        out_type=[
            jax.ShapeDtypeStruct((_NW, _NCHUNK * _CAP), jnp.int32),
            jax.ShapeDtypeStruct((_NW, _NCHUNK * _CAP), jnp.int32),
            jax.ShapeDtypeStruct((_NW, _NCHUNK * _CAP), jnp.float32),
            jax.ShapeDtypeStruct((_NW, 16), jnp.int32),
            jax.ShapeDtypeStruct((_NW, _NCHUNK, _CHB), jnp.float32),
            jax.ShapeDtypeStruct((_NCHUNK, 16, _NW, _S2CAP), jnp.int32),
            jax.ShapeDtypeStruct((_NCHUNK, 16, _NW, _S2CAP), jnp.int32),
            jax.ShapeDtypeStruct((_NCHUNK, 16, _NW, _S2CAP), jnp.float32),
            jax.ShapeDtypeStruct((_NCHUNK, _NW, 16), jnp.int32),
        ],
        mesh=_sc_mesh(),
        compiler_params=pltpu.CompilerParams(needs_layout_passes=False),
        scratch_types=[
            pltpu.VMEM((_EH,), jnp.int32),
            pltpu.VMEM((_EH,), jnp.int32),
            pltpu.VMEM((_EH,), jnp.float32),
            pltpu.VMEM((_NCHUNK * _CAP,), jnp.int32),
            pltpu.VMEM((_NCHUNK * _CAP,), jnp.int32),
            pltpu.VMEM((_NCHUNK * _CAP,), jnp.float32),
            pltpu.VMEM((_CHB,), jnp.float32),
            pltpu.VMEM((16,), jnp.int32),
            pltpu.VMEM((_S2CAP * 16,), jnp.int32),
            pltpu.VMEM((_S2CAP * 16,), jnp.int32),
            pltpu.VMEM((_S2CAP * 16,), jnp.float32),
            pltpu.VMEM((16,), jnp.int32),
        ],
    )
    return f(src, dst, ew)


def _pb_body(y_hbm, bsrc_hbm, bdl_hbm, bew_hbm, pyp_hbm,
             y_v, src_v, dl_v, ew_v, yacc_v):
    wid = _wid()
    pltpu.sync_copy(y_hbm, y_v)
    for c in range(_NCHUNK):
        _zero_ref(yacc_v, _CHB, jnp.float32)
        pltpu.sync_copy(bsrc_hbm.at[wid, c], src_v)
        pltpu.sync_copy(bdl_hbm.at[wid, c], dl_v)
        pltpu.sync_copy(bew_hbm.at[wid, c], ew_v)

        def body(i, _):
            s16 = src_v[pl.ds(i * 16, 16)]
            dl16 = dl_v[pl.ds(i * 16, 16)]
            w16 = ew_v[pl.ds(i * 16, 16)]
            vals = plsc.load_gather(y_v, [s16])
            plsc.addupdate_scatter(yacc_v, [dl16], vals * w16)
            return 0

        lax.fori_loop(0, _CAP // 16, body, 0)
        pltpu.sync_copy(yacc_v, pyp_hbm.at[wid, c])


def _run_pb(y, bsrc, bdl, bew):
    f = pl.kernel(
        _pb_body,
        out_type=jax.ShapeDtypeStruct((_NW, _NCHUNK, _CHB), jnp.float32),
        mesh=_sc_mesh(),
        compiler_params=pltpu.CompilerParams(needs_layout_passes=False),
        scratch_types=[
            pltpu.VMEM((_N,), jnp.float32),
            pltpu.VMEM((_CAP,), jnp.int32),
            pltpu.VMEM((_CAP,), jnp.int32),
            pltpu.VMEM((_CAP,), jnp.float32),
            pltpu.VMEM((_CHB,), jnp.float32),
        ],
    )
    return f(y, bsrc, bdl, bew)


def _pc_body(g_hbm, bsrc2_hbm, bdl2_hbm, bew2_hbm, cnts2_hbm, out_hbm,
             sraw_v, draw_v, wraw_v, cslab_v, src_c, dl_c, ew_c,
             rows0_v, rows1_v, acc_v, sem0, sem1):
    core = lax.axis_index("c")
    sub = lax.axis_index("s")
    iota = lax.iota(jnp.int32, 16)
    zf = jnp.zeros((16,), jnp.float32)
    zi = jnp.zeros((16,), jnp.int32)
    cols = [iota + j * 16 for j in range(16)]

    def run_chunk(cc, _):
        chunk = core * 8 + cc

        def za(i, _2):
            for j in range(16):
                acc_v[i, pl.ds(j * 16, 16)] = zf
            return 0

        lax.fori_loop(0, 256, za, 0)

        pltpu.sync_copy(bsrc2_hbm.at[chunk, sub], sraw_v)
        pltpu.sync_copy(bdl2_hbm.at[chunk, sub], draw_v)
        pltpu.sync_copy(bew2_hbm.at[chunk, sub], wraw_v)
        pltpu.sync_copy(cnts2_hbm.at[chunk], cslab_v)

        cur = jnp.int32(0)
        for w in range(_NW):
            crow = cslab_v[w, pl.ds(0, 16)]
            c_w = jnp.max(jnp.where(iota == sub, crow, 0))
            for j in range(_S2CAP // 16):
                m = (cols[j] if j < 16 else iota + j * 16) < c_w
                plsc.store_compressed(src_c.at[pl.ds(cur, 16)],
                                      sraw_v[w, pl.ds(j * 16, 16)], mask=m)
                plsc.store_compressed(dl_c.at[pl.ds(cur, 16)],
                                      draw_v[w, pl.ds(j * 16, 16)], mask=m)
                plsc.store_compressed(ew_c.at[pl.ds(cur, 16)],
                                      wraw_v[w, pl.ds(j * 16, 16)], mask=m)
                take = jnp.clip(c_w - j * 16, 0, 16)
                cur = cur + take
        ncomp = cur
        t0 = (ncomp // 16) * 16
        for k in range(5):
            pos = t0 + k * 16
            m = (iota + pos) >= ncomp
            sv = src_c[pl.ds(pos, 16)]
            src_c[pl.ds(pos, 16)] = jnp.where(m, 0, sv)
            wv = ew_c[pl.ds(pos, 16)]
            ew_c[pl.ds(pos, 16)] = jnp.where(m, 0.0, wv)
            dv = dl_c[pl.ds(pos, 16)]
            dl_c[pl.ds(pos, 16)] = jnp.where(m, 0, dv)
        nb = (ncomp + 63) // 64

        def issue(b, rows_v, sem):
            return pltpu.async_copy(
                g_hbm.at[src_c.at[pl.ds(b * 64, 64)]], rows_v, sem)

        def process(b, rows_v):
            def grp(g, _3):
                w16 = ew_c[pl.ds(b * 64 + g * 16, 16)]
                dl16 = lax.bitwise_and(dl_c[pl.ds(b * 64 + g * 16, 16)], 255)
                for r in range(16):
                    e = g * 16 + r
                    w = w16[r]
                    row16 = iota * 0 + dl16[r]
                    for j in range(16):
                        v = rows_v[e, pl.ds(j * 16, 16)] * w
                        plsc.addupdate_scatter(acc_v, [row16, cols[j]], v)
                return 0

            lax.fori_loop(0, 4, grp, 0)

        @pl.when(nb > 0)
        def _prime():
            issue(0, rows0_v, sem0)

        def pair(p, _2):
            b0 = 2 * p
            b1 = b0 + 1
            pltpu.make_async_copy(
                g_hbm.at[src_c.at[pl.ds(b0 * 64, 64)]], rows0_v, sem0).wait()

            @pl.when(b1 < nb)
            def _i1():
                issue(b1, rows1_v, sem1)

            process(b0, rows0_v)

            @pl.when(b1 < nb)
            def _p1():
                pltpu.make_async_copy(
                    g_hbm.at[src_c.at[pl.ds(b1 * 64, 64)]], rows1_v,
                    sem1).wait()

                @pl.when(b1 + 1 < nb)
                def _i2():
                    issue(b1 + 1, rows0_v, sem0)

                process(b1, rows1_v)

            return 0

        lax.fori_loop(0, (nb + 1) // 2, pair, 0)

        off = chunk * _CHB + sub * 256

        @pl.when(off < _N)
        def _wb():
            pltpu.sync_copy(acc_v, out_hbm.at[pl.ds(off, 256)])

        return 0

    lax.fori_loop(0, 8, run_chunk, 0)


def _run_pc(g, bsrc2, bdl2, bew2, cnts2):
    f = pl.kernel(
        _pc_body,
        out_type=jax.ShapeDtypeStruct((_N, 256), jnp.float32),
        mesh=_sc_mesh(),
        compiler_params=pltpu.CompilerParams(needs_layout_passes=False),
        scratch_types=[
            pltpu.VMEM((_NW, _S2CAP), jnp.int32),
            pltpu.VMEM((_NW, _S2CAP), jnp.int32),
            pltpu.VMEM((_NW, _S2CAP), jnp.float32),
            pltpu.VMEM((_NW, 16), jnp.int32),
            pltpu.VMEM((_NW * _S2CAP + 80,), jnp.int32),
            pltpu.VMEM((_NW * _S2CAP + 80,), jnp.int32),
            pltpu.VMEM((_NW * _S2CAP + 80,), jnp.float32),
            pltpu.VMEM((64, 256), jnp.float32),
            pltpu.VMEM((64, 256), jnp.float32),
            pltpu.VMEM((256, 256), jnp.float32),
            pltpu.SemaphoreType.DMA,
            pltpu.SemaphoreType.DMA,
        ],
    )
    return f(g, bsrc2, bdl2, bew2, cnts2)


def _mm_body(h_ref, w_ref, al_ref, be_ref, dv_ref, b_ref, o_ref, acc_ref,
             *, act, affine, rowscale):
    k = pl.program_id(2)

    @pl.when(k == 0)
    def _init():
        acc_ref[...] = jnp.zeros_like(acc_ref)

    h = h_ref[...]
    if affine:
        h = h * al_ref[...] + be_ref[...]
    acc_ref[...] += jnp.dot(h, w_ref[...], preferred_element_type=jnp.float32)

    @pl.when(k == pl.num_programs(2) - 1)
    def _fin():
        r = acc_ref[...] + b_ref[...]
        if rowscale:
            r = r * dv_ref[...]
        if act:
            r = jnp.maximum(r, 0.0)
        o_ref[...] = r


def _matmul(h, W, b, bm, bn, bk, act=False, affine=None, dinv=None):
    M, K = h.shape
    K2, Nn = W.shape
    assert K == K2 and M % bm == 0 and Nn % bn == 0 and K % bk == 0
    if affine is None:
        al = jnp.ones((1, K), jnp.float32)
        bt = jnp.zeros((1, K), jnp.float32)
    else:
        al, bt = affine
        al = al.reshape(1, K)
        bt = bt.reshape(1, K)
    dv = jnp.ones((M, 1), jnp.float32) if dinv is None else dinv.reshape(M, 1)
    return pl.pallas_call(
        functools.partial(_mm_body, act=act, affine=affine is not None,
                          rowscale=dinv is not None),
        grid=(M // bm, Nn // bn, K // bk),
        in_specs=[
            pl.BlockSpec((bm, bk), lambda m, i, k: (m, k)),
            pl.BlockSpec((bk, bn), lambda m, i, k: (k, i)),
            pl.BlockSpec((1, bk), lambda m, i, k: (0, k)),
            pl.BlockSpec((1, bk), lambda m, i, k: (0, k)),
            pl.BlockSpec((bm, 1), lambda m, i, k: (m, 0)),
            pl.BlockSpec((1, bn), lambda m, i, k: (0, i)),
        ],
        out_specs=pl.BlockSpec((bm, bn), lambda m, i, k: (m, i)),
        out_shape=jax.ShapeDtypeStruct((M, Nn), jnp.float32),
        scratch_shapes=[pltpu.VMEM((bm, bn), jnp.float32)],
        compiler_params=pltpu.CompilerParams(
            dimension_semantics=("parallel", "parallel", "arbitrary"),
        ),
    )(h, W, al, bt, dv, b.reshape(1, -1))


def _stats_body(s_ref, g_ref, dv_ref, b_ref, r_ref, st_ref, sacc_ref):
    i = pl.program_id(0)

    @pl.when(i == 0)
    def _init():
        sacc_ref[...] = jnp.zeros_like(sacc_ref)

    r = jnp.maximum(dv_ref[...] * (s_ref[...] + g_ref[...]) + b_ref[...],
                    0.0)
    r_ref[...] = r
    sacc_ref[0, :] += jnp.sum(r, axis=0)
    sacc_ref[1, :] += jnp.sum(r * r, axis=0)

    @pl.when(i == pl.num_programs(0) - 1)
    def _fin():
        st_ref[...] = sacc_ref[...]


def _relu_stats(s, g, dinv, b, bm=3968):
    M = s.shape[0]
    return pl.pallas_call(
        _stats_body,
        grid=(M // bm,),
        in_specs=[
            pl.BlockSpec((bm, 256), lambda i: (i, 0)),
            pl.BlockSpec((bm, 256), lambda i: (i, 0)),
            pl.BlockSpec((bm, 1), lambda i: (i, 0)),
            pl.BlockSpec((1, 256), lambda i: (0, 0)),
        ],
        out_specs=[
            pl.BlockSpec((bm, 256), lambda i: (i, 0)),
            pl.BlockSpec((2, 256), lambda i: (0, 0)),
        ],
        out_shape=[
            jax.ShapeDtypeStruct((M, 256), jnp.float32),
            jax.ShapeDtypeStruct((2, 256), jnp.float32),
        ],
        scratch_shapes=[pltpu.VMEM((2, 256), jnp.float32)],
        compiler_params=pltpu.CompilerParams(
            dimension_semantics=("arbitrary",),
        ),
    )(s, g, dinv.reshape(M, 1), b.reshape(1, 256))


def _l1_body(ax_ref, w_ref, b_ref, r_ref, st_ref, sacc_ref):
    i = pl.program_id(0)

    @pl.when(i == 0)
    def _init():
        sacc_ref[...] = jnp.zeros_like(sacc_ref)

    r = jnp.maximum(ax_ref[...] * w_ref[...] + b_ref[...], 0.0)
    r_ref[...] = r
    sacc_ref[0, :] += jnp.sum(r, axis=0)
    sacc_ref[1, :] += jnp.sum(r * r, axis=0)

    @pl.when(i == pl.num_programs(0) - 1)
    def _fin():
        st_ref[...] = sacc_ref[...]


def _l1_relu_stats(ax, w0, b0, bm=3968):
    M = ax.shape[0]
    return pl.pallas_call(
        _l1_body,
        grid=(M // bm,),
        in_specs=[
            pl.BlockSpec((bm, 1), lambda i: (i, 0)),
            pl.BlockSpec((1, 256), lambda i: (0, 0)),
            pl.BlockSpec((1, 256), lambda i: (0, 0)),
        ],
        out_specs=[
            pl.BlockSpec((bm, 256), lambda i: (i, 0)),
            pl.BlockSpec((2, 256), lambda i: (0, 0)),
        ],
        out_shape=[
            jax.ShapeDtypeStruct((M, 256), jnp.float32),
            jax.ShapeDtypeStruct((2, 256), jnp.float32),
        ],
        scratch_shapes=[pltpu.VMEM((2, 256), jnp.float32)],
        compiler_params=pltpu.CompilerParams(
            dimension_semantics=("arbitrary",),
        ),
    )(ax.reshape(M, 1), w0.reshape(1, 256), b0.reshape(1, 256))


def _affine_from_stats(st, g, be):
    mu = st[0] / _N
    var = st[1] / _N - mu * mu
    al = g * lax.rsqrt(var + _EPS)
    return al, be - mu * al


def kernel(x, edge_index, edge_weights, W0, b0, g0, be0, W1, b1, g1, be1,
           W2, b2, g2, be2, Wf1, bf1, Wf2, bf2, Wo, bo):
    src = edge_index[0]
    dst = edge_index[1]

    (bsrc, bdl, bew, cnts, degp,
     bsrc2, bdl2, bew2, cnts2) = _run_pa(src, dst, edge_weights)
    bsrc3 = bsrc.reshape(_NW, _NCHUNK, _CAP)
    bdl3 = bdl.reshape(_NW, _NCHUNK, _CAP)
    bew3 = bew.reshape(_NW, _NCHUNK, _CAP)

    deg = degp.reshape(_NW, _NCHUNK * _CHB).sum(axis=0)[:_N] + 1.0
    dinv = lax.rsqrt(deg)
    y = dinv * x[:, 0]

    pyp = _run_pb(y, bsrc3, bdl3, bew3)
    sy = pyp.reshape(_NW, _NCHUNK * _CHB).sum(axis=0)[:_N]
    ax = dinv * (sy + y)

    r1, st1 = _l1_relu_stats(ax, W0[0], b0)
    al1, bt1 = _affine_from_stats(st1, g0, be0)

    G2 = _matmul(r1, W1, jnp.zeros((256,), jnp.float32), bm=3968, bn=256,
                 bk=256, affine=(al1, bt1), dinv=dinv)
    s2 = _run_pc(G2, bsrc2, bdl2, bew2, cnts2)
    r2, st2 = _relu_stats(s2, G2, dinv, b1)
    al2, bt2 = _affine_from_stats(st2, g1, be1)

    G3 = _matmul(r2, W2, jnp.zeros((256,), jnp.float32), bm=3968, bn=256,
                 bk=256, affine=(al2, bt2), dinv=dinv)
    s3 = _run_pc(G3, bsrc2, bdl2, bew2, cnts2)
    r3, st3 = _relu_stats(s3, G3, dinv, b2)
    al3, bt3 = _affine_from_stats(st3, g2, be2)

    h = r3.reshape(_B, _NPB * 256)
    alf = jnp.tile(al3, _NPB)
    btf = jnp.tile(bt3, _NPB)
    h = _matmul(h, Wf1, bf1, bm=_B, bn=1024, bk=512, act=True,
                affine=(alf, btf))
    h = _matmul(h, Wf2, bf2, bm=_B, bn=1024, bk=512, act=True)
    return _matmul(h, Wo, bo, bm=_B, bn=256, bk=512)


# speedup vs baseline: 1.1107x; 1.0004x over previous
import functools

import jax
import jax.numpy as jnp
from jax import lax
from jax.experimental import pallas as pl
from jax.experimental.pallas import tpu as pltpu
from jax.experimental.pallas import tpu_sc as plsc

_B = 1024
_NPB = 62
_N = _B * _NPB
_E = 317440
_NW = 32
_EPW = _E // _NW
_EH = _EPW // 2
_NCHUNK = 16
_CHB = 4096
_CAP = 2048
_S2CAP = 128
_EPS = 1e-5


def _sc_mesh():
    return plsc.VectorSubcoreMesh(core_axis_name="c", subcore_axis_name="s",
                                  num_cores=2, num_subcores=16)


def _wid():
    return lax.axis_index("s") * 2 + lax.axis_index("c")


def _zero_ref(ref, n, dtype):
    z = jnp.zeros((16,), dtype)

    def body(i, _):
        ref[pl.ds(i * 16, 16)] = z
        return 0

    lax.fori_loop(0, n // 16, body, 0)


def _pa_body(src_hbm, dst_hbm, ew_hbm,
             bsrc_hbm, bdl_hbm, bew_hbm, cnts_hbm, degp_hbm,
             bsrc2_hbm, bdl2_hbm, bew2_hbm, cnts2_hbm,
             es_v, ed_v, ee_v, bsrc_v, bdl_v, bew_v, dacc_v, cnt_v,
             ss_v, sd_v, sw_v, cnt2_v):
    wid = _wid()
    iota = lax.iota(jnp.int32, 16)
    zi = jnp.zeros((16,), jnp.int32)
    zf = jnp.zeros((16,), jnp.float32)

    def zb(i, _):
        bsrc_v[pl.ds(i * 16, 16)] = zi
        bdl_v[pl.ds(i * 16, 16)] = zi
        bew_v[pl.ds(i * 16, 16)] = zf
        return 0

    lax.fori_loop(0, _NCHUNK * _CAP // 16, zb, 0)

    base = wid * _EPW
    curs = tuple(jnp.int32(0) for _ in range(_NCHUNK))
    for h in range(2):
        pltpu.sync_copy(src_hbm.at[pl.ds(base + h * _EH, _EH)], es_v)
        pltpu.sync_copy(dst_hbm.at[pl.ds(base + h * _EH, _EH)], ed_v)
        pltpu.sync_copy(ew_hbm.at[pl.ds(base + h * _EH, _EH)], ee_v)

        def vbody(i, cs):
            s16 = es_v[pl.ds(i * 16, 16)]
            d16 = ed_v[pl.ds(i * 16, 16)]
            w16 = ee_v[pl.ds(i * 16, 16)]
            ch = lax.shift_right_logical(d16, 12)
            dl = lax.bitwise_and(d16, 4095)
            new = []
            for c in range(_NCHUNK):
                m = ch == c
                pc = jnp.sum(m.astype(jnp.int32))
                cur = jnp.minimum(cs[c], _CAP - 16)
                off = cur + c * _CAP
                plsc.store_compressed(bsrc_v.at[pl.ds(off, 16)], s16,
                                      mask=m)
                plsc.store_compressed(bdl_v.at[pl.ds(off, 16)], dl,
                                      mask=m)
                plsc.store_compressed(bew_v.at[pl.ds(off, 16)], w16,
                                      mask=m)
                new.append(cur + pc)
            return tuple(new)

        curs = lax.fori_loop(0, _EH // 16, vbody, curs)

    cvec = jnp.zeros((16,), jnp.int32)
    for c in range(_NCHUNK):
        cvec = cvec + jnp.where(iota == c, curs[c], 0)
    cnt_v[...] = cvec
    pltpu.sync_copy(cnt_v, cnts_hbm.at[wid])
    pltpu.sync_copy(bsrc_v, bsrc_hbm.at[wid])
    pltpu.sync_copy(bdl_v, bdl_hbm.at[wid])
    pltpu.sync_copy(bew_v, bew_hbm.at[wid])

    for c in range(_NCHUNK):
        _zero_ref(dacc_v, _CHB, jnp.float32)

        def dbody(i, _):
            dl16 = bdl_v[pl.ds(c * _CAP + i * 16, 16)]
            w16 = bew_v[pl.ds(c * _CAP + i * 16, 16)]
            plsc.addupdate_scatter(dacc_v, [dl16], w16)
            return 0

        lax.fori_loop(0, _CAP // 16, dbody, 0)
        pltpu.sync_copy(dacc_v, degp_hbm.at[wid, c])

    zi16 = jnp.zeros((16,), jnp.int32)
    zf16 = jnp.zeros((16,), jnp.float32)

    def s2_chunk(c, _):
        def z2(i, _2):
            ss_v[pl.ds(i * 16, 16)] = zi16
            sd_v[pl.ds(i * 16, 16)] = zi16
            sw_v[pl.ds(i * 16, 16)] = zf16
            return 0

        lax.fori_loop(0, _S2CAP * 16 // 16, z2, 0)
        cnt_c = jnp.max(jnp.where(iota == c, cvec, 0))
        nv = (cnt_c + 15) // 16

        def s2v(i, cs2):
            base_e = c * _CAP + i * 16
            s16 = bsrc_v[pl.ds(base_e, 16)]
            d16 = bdl_v[pl.ds(base_e, 16)]
            w16 = bew_v[pl.ds(base_e, 16)]
            valid = (iota + i * 16) < cnt_c
            sb = lax.shift_right_logical(d16, 8)
            new = []
            for s in range(16):
                m = (sb == s) & valid
                pc = jnp.sum(m.astype(jnp.int32))
                cur = jnp.minimum(cs2[s], _S2CAP - 16)
                off = s * _S2CAP + cur
                plsc.store_compressed(ss_v.at[pl.ds(off, 16)], s16, mask=m)
                plsc.store_compressed(sd_v.at[pl.ds(off, 16)], d16, mask=m)
                plsc.store_compressed(sw_v.at[pl.ds(off, 16)], w16, mask=m)
                new.append(cur + pc)
            return tuple(new)

        cs2 = lax.fori_loop(0, nv, s2v,
                            tuple(jnp.int32(0) for _ in range(16)))
        c2v = jnp.zeros((16,), jnp.int32)
        for s in range(16):
            c2v = c2v + jnp.where(iota == s, cs2[s], 0)
        cnt2_v[...] = c2v
        pltpu.sync_copy(cnt2_v, cnts2_hbm.at[c, wid])
        for s in range(16):
            pltpu.sync_copy(ss_v.at[pl.ds(s * _S2CAP, _S2CAP)],
                            bsrc2_hbm.at[c, s, wid])
            pltpu.sync_copy(sd_v.at[pl.ds(s * _S2CAP, _S2CAP)],
                            bdl2_hbm.at[c, s, wid])
            pltpu.sync_copy(sw_v.at[pl.ds(s * _S2CAP, _S2CAP)],
                            bew2_hbm.at[c, s, wid])
        return 0

    lax.fori_loop(0, _NCHUNK, s2_chunk, 0)


def _run_pa(src, dst, ew):
    f = pl.kernel(
        _pa_body,
        out_type=[
            jax.ShapeDtypeStruct((_NW, _NCHUNK * _CAP), jnp.int32),
            jax.ShapeDtypeStruct((_NW, _NCHUNK * _CAP), jnp.int32),
            jax.ShapeDtypeStruct((_NW, _NCHUNK * _CAP), jnp.float32),
            jax.ShapeDtypeStruct((_NW, 16), jnp.int32),
            jax.ShapeDtypeStruct((_NW, _NCHUNK, _CHB), jnp.float32),
            jax.ShapeDtypeStruct((_NCHUNK, 16, _NW, _S2CAP), jnp.int32),
            jax.ShapeDtypeStruct((_NCHUNK, 16, _NW, _S2CAP), jnp.int32),
            jax.ShapeDtypeStruct((_NCHUNK, 16, _NW, _S2CAP), jnp.float32),
            jax.ShapeDtypeStruct((_NCHUNK, _NW, 16), jnp.int32),
        ],
        mesh=_sc_mesh(),
        compiler_params=pltpu.CompilerParams(needs_layout_passes=False),
        scratch_types=[
            pltpu.VMEM((_EH,), jnp.int32),
            pltpu.VMEM((_EH,), jnp.int32),
            pltpu.VMEM((_EH,), jnp.float32),
            pltpu.VMEM((_NCHUNK * _CAP,), jnp.int32),
            pltpu.VMEM((_NCHUNK * _CAP,), jnp.int32),
            pltpu.VMEM((_NCHUNK * _CAP,), jnp.float32),
            pltpu.VMEM((_CHB,), jnp.float32),
            pltpu.VMEM((16,), jnp.int32),
            pltpu.VMEM((_S2CAP * 16,), jnp.int32),
            pltpu.VMEM((_S2CAP * 16,), jnp.int32),
            pltpu.VMEM((_S2CAP * 16,), jnp.float32),
            pltpu.VMEM((16,), jnp.int32),
        ],
    )
    return f(src, dst, ew)


def _pb_body(y_hbm, bsrc_hbm, bdl_hbm, bew_hbm, pyp_hbm,
             y_v, src_v, dl_v, ew_v, yacc_v):
    wid = _wid()
    pltpu.sync_copy(y_hbm, y_v)
    for c in range(_NCHUNK):
        _zero_ref(yacc_v, _CHB, jnp.float32)
        pltpu.sync_copy(bsrc_hbm.at[wid, c], src_v)
        pltpu.sync_copy(bdl_hbm.at[wid, c], dl_v)
        pltpu.sync_copy(bew_hbm.at[wid, c], ew_v)

        def body(i, _):
            s16 = src_v[pl.ds(i * 16, 16)]
            dl16 = dl_v[pl.ds(i * 16, 16)]
            w16 = ew_v[pl.ds(i * 16, 16)]
            vals = plsc.load_gather(y_v, [s16])
            plsc.addupdate_scatter(yacc_v, [dl16], vals * w16)
            return 0

        lax.fori_loop(0, _CAP // 16, body, 0)
        pltpu.sync_copy(yacc_v, pyp_hbm.at[wid, c])


def _run_pb(y, bsrc, bdl, bew):
    f = pl.kernel(
        _pb_body,
        out_type=jax.ShapeDtypeStruct((_NW, _NCHUNK, _CHB), jnp.float32),
        mesh=_sc_mesh(),
        compiler_params=pltpu.CompilerParams(needs_layout_passes=False),
        scratch_types=[
            pltpu.VMEM((_N,), jnp.float32),
            pltpu.VMEM((_CAP,), jnp.int32),
            pltpu.VMEM((_CAP,), jnp.int32),
            pltpu.VMEM((_CAP,), jnp.float32),
            pltpu.VMEM((_CHB,), jnp.float32),
        ],
    )
    return f(y, bsrc, bdl, bew)


def _pc_body(g_hbm, bsrc2_hbm, bdl2_hbm, bew2_hbm, cnts2_hbm, out_hbm,
             sraw_v, draw_v, wraw_v, cslab_v, src_c, dl_c, ew_c,
             rows0_v, rows1_v, acc_v, sem0, sem1):
    core = lax.axis_index("c")
    sub = lax.axis_index("s")
    iota = lax.iota(jnp.int32, 16)
    zf = jnp.zeros((16,), jnp.float32)
    zi = jnp.zeros((16,), jnp.int32)
    cols = [iota + j * 16 for j in range(16)]

    def run_chunk(cc, _):
        chunk = core * 8 + cc

        def za(i, _2):
            for j in range(16):
                acc_v[i, pl.ds(j * 16, 16)] = zf
            return 0

        lax.fori_loop(0, 256, za, 0)

        pltpu.sync_copy(bsrc2_hbm.at[chunk, sub], sraw_v)
        pltpu.sync_copy(bdl2_hbm.at[chunk, sub], draw_v)
        pltpu.sync_copy(bew2_hbm.at[chunk, sub], wraw_v)
        pltpu.sync_copy(cnts2_hbm.at[chunk], cslab_v)

        cur = jnp.int32(0)
        for w in range(_NW):
            crow = cslab_v[w, pl.ds(0, 16)]
            c_w = jnp.max(jnp.where(iota == sub, crow, 0))
            for j in range(_S2CAP // 16):
                m = (cols[j] if j < 16 else iota + j * 16) < c_w
                plsc.store_compressed(src_c.at[pl.ds(cur, 16)],
                                      sraw_v[w, pl.ds(j * 16, 16)], mask=m)
                plsc.store_compressed(dl_c.at[pl.ds(cur, 16)],
                                      draw_v[w, pl.ds(j * 16, 16)], mask=m)
                plsc.store_compressed(ew_c.at[pl.ds(cur, 16)],
                                      wraw_v[w, pl.ds(j * 16, 16)], mask=m)
                take = jnp.clip(c_w - j * 16, 0, 16)
                cur = cur + take
        ncomp = cur
        t0 = (ncomp // 16) * 16
        for k in range(5):
            pos = t0 + k * 16
            m = (iota + pos) >= ncomp
            sv = src_c[pl.ds(pos, 16)]
            src_c[pl.ds(pos, 16)] = jnp.where(m, 0, sv)
            wv = ew_c[pl.ds(pos, 16)]
            ew_c[pl.ds(pos, 16)] = jnp.where(m, 0.0, wv)
            dv = dl_c[pl.ds(pos, 16)]
            dl_c[pl.ds(pos, 16)] = jnp.where(m, 0, dv)
        nb = (ncomp + 63) // 64

        def issue(b, rows_v, sem):
            return pltpu.async_copy(
                g_hbm.at[src_c.at[pl.ds(b * 64, 64)]], rows_v, sem)

        def process(b, rows_v):
            def grp(g, _3):
                w16 = ew_c[pl.ds(b * 64 + g * 16, 16)]
                dl16 = lax.bitwise_and(dl_c[pl.ds(b * 64 + g * 16, 16)], 255)
                for r in range(16):
                    e = g * 16 + r
                    w = w16[r]
                    row16 = iota * 0 + dl16[r]
                    for j in range(16):
                        v = rows_v[e, pl.ds(j * 16, 16)] * w
                        plsc.addupdate_scatter(acc_v, [row16, cols[j]], v)
                return 0

            lax.fori_loop(0, 4, grp, 0)

        @pl.when(nb > 0)
        def _prime():
            issue(0, rows0_v, sem0)

        def pair(p, _2):
            b0 = 2 * p
            b1 = b0 + 1
            pltpu.make_async_copy(
                g_hbm.at[src_c.at[pl.ds(b0 * 64, 64)]], rows0_v, sem0).wait()

            @pl.when(b1 < nb)
            def _i1():
                issue(b1, rows1_v, sem1)

            process(b0, rows0_v)

            @pl.when(b1 < nb)
            def _p1():
                pltpu.make_async_copy(
                    g_hbm.at[src_c.at[pl.ds(b1 * 64, 64)]], rows1_v,
                    sem1).wait()

                @pl.when(b1 + 1 < nb)
                def _i2():
                    issue(b1 + 1, rows0_v, sem0)

                process(b1, rows1_v)

            return 0

        lax.fori_loop(0, (nb + 1) // 2, pair, 0)

        off = chunk * _CHB + sub * 256

        @pl.when(off < _N)
        def _wb():
            pltpu.sync_copy(acc_v, out_hbm.at[pl.ds(off, 256)])

        return 0

    lax.fori_loop(0, 8, run_chunk, 0)


def _run_pc(g, bsrc2, bdl2, bew2, cnts2):
    f = pl.kernel(
        _pc_body,
        out_type=jax.ShapeDtypeStruct((_N, 256), jnp.float32),
        mesh=_sc_mesh(),
        compiler_params=pltpu.CompilerParams(needs_layout_passes=False),
        scratch_types=[
            pltpu.VMEM((_NW, _S2CAP), jnp.int32),
            pltpu.VMEM((_NW, _S2CAP), jnp.int32),
            pltpu.VMEM((_NW, _S2CAP), jnp.float32),
            pltpu.VMEM((_NW, 16), jnp.int32),
            pltpu.VMEM((_NW * _S2CAP + 80,), jnp.int32),
            pltpu.VMEM((_NW * _S2CAP + 80,), jnp.int32),
            pltpu.VMEM((_NW * _S2CAP + 80,), jnp.float32),
            pltpu.VMEM((64, 256), jnp.float32),
            pltpu.VMEM((64, 256), jnp.float32),
            pltpu.VMEM((256, 256), jnp.float32),
            pltpu.SemaphoreType.DMA,
            pltpu.SemaphoreType.DMA,
        ],
    )
    return f(g, bsrc2, bdl2, bew2, cnts2)


def _mm_body(h_ref, w_ref, al_ref, be_ref, dv_ref, b_ref, o_ref, acc_ref,
             *, act, affine, rowscale):
    k = pl.program_id(2)

    @pl.when(k == 0)
    def _init():
        acc_ref[...] = jnp.zeros_like(acc_ref)

    h = h_ref[...]
    if affine:
        h = h * al_ref[...] + be_ref[...]
    acc_ref[...] += jnp.dot(h.astype(jnp.bfloat16),
                            w_ref[...].astype(jnp.bfloat16),
                            preferred_element_type=jnp.float32)

    @pl.when(k == pl.num_programs(2) - 1)
    def _fin():
        r = acc_ref[...] + b_ref[...]
        if rowscale:
            r = r * dv_ref[...]
        if act:
            r = jnp.maximum(r, 0.0)
        o_ref[...] = r


def _matmul(h, W, b, bm, bn, bk, act=False, affine=None, dinv=None):
    M, K = h.shape
    K2, Nn = W.shape
    assert K == K2 and M % bm == 0 and Nn % bn == 0 and K % bk == 0
    if affine is None:
        al = jnp.ones((1, K), jnp.float32)
        bt = jnp.zeros((1, K), jnp.float32)
    else:
        al, bt = affine
        al = al.reshape(1, K)
        bt = bt.reshape(1, K)
    dv = jnp.ones((M, 1), jnp.float32) if dinv is None else dinv.reshape(M, 1)
    return pl.pallas_call(
        functools.partial(_mm_body, act=act, affine=affine is not None,
                          rowscale=dinv is not None),
        grid=(M // bm, Nn // bn, K // bk),
        in_specs=[
            pl.BlockSpec((bm, bk), lambda m, i, k: (m, k)),
            pl.BlockSpec((bk, bn), lambda m, i, k: (k, i)),
            pl.BlockSpec((1, bk), lambda m, i, k: (0, k)),
            pl.BlockSpec((1, bk), lambda m, i, k: (0, k)),
            pl.BlockSpec((bm, 1), lambda m, i, k: (m, 0)),
            pl.BlockSpec((1, bn), lambda m, i, k: (0, i)),
        ],
        out_specs=pl.BlockSpec((bm, bn), lambda m, i, k: (m, i)),
        out_shape=jax.ShapeDtypeStruct((M, Nn), jnp.float32),
        scratch_shapes=[pltpu.VMEM((bm, bn), jnp.float32)],
        compiler_params=pltpu.CompilerParams(
            dimension_semantics=("parallel", "parallel", "arbitrary"),
        ),
    )(h, W, al, bt, dv, b.reshape(1, -1))


def _stats_body(s_ref, g_ref, dv_ref, b_ref, r_ref, st_ref, sacc_ref):
    i = pl.program_id(0)

    @pl.when(i == 0)
    def _init():
        sacc_ref[...] = jnp.zeros_like(sacc_ref)

    r = jnp.maximum(dv_ref[...] * (s_ref[...] + g_ref[...]) + b_ref[...],
                    0.0)
    r_ref[...] = r
    sacc_ref[0, :] += jnp.sum(r, axis=0)
    sacc_ref[1, :] += jnp.sum(r * r, axis=0)

    @pl.when(i == pl.num_programs(0) - 1)
    def _fin():
        st_ref[...] = sacc_ref[...]


def _relu_stats(s, g, dinv, b, bm=3968):
    M = s.shape[0]
    return pl.pallas_call(
        _stats_body,
        grid=(M // bm,),
        in_specs=[
            pl.BlockSpec((bm, 256), lambda i: (i, 0)),
            pl.BlockSpec((bm, 256), lambda i: (i, 0)),
            pl.BlockSpec((bm, 1), lambda i: (i, 0)),
            pl.BlockSpec((1, 256), lambda i: (0, 0)),
        ],
        out_specs=[
            pl.BlockSpec((bm, 256), lambda i: (i, 0)),
            pl.BlockSpec((2, 256), lambda i: (0, 0)),
        ],
        out_shape=[
            jax.ShapeDtypeStruct((M, 256), jnp.float32),
            jax.ShapeDtypeStruct((2, 256), jnp.float32),
        ],
        scratch_shapes=[pltpu.VMEM((2, 256), jnp.float32)],
        compiler_params=pltpu.CompilerParams(
            dimension_semantics=("arbitrary",),
        ),
    )(s, g, dinv.reshape(M, 1), b.reshape(1, 256))


def _l1_body(ax_ref, w_ref, b_ref, r_ref, st_ref, sacc_ref):
    i = pl.program_id(0)

    @pl.when(i == 0)
    def _init():
        sacc_ref[...] = jnp.zeros_like(sacc_ref)

    r = jnp.maximum(ax_ref[...] * w_ref[...] + b_ref[...], 0.0)
    r_ref[...] = r
    sacc_ref[0, :] += jnp.sum(r, axis=0)
    sacc_ref[1, :] += jnp.sum(r * r, axis=0)

    @pl.when(i == pl.num_programs(0) - 1)
    def _fin():
        st_ref[...] = sacc_ref[...]


def _l1_relu_stats(ax, w0, b0, bm=3968):
    M = ax.shape[0]
    return pl.pallas_call(
        _l1_body,
        grid=(M // bm,),
        in_specs=[
            pl.BlockSpec((bm, 1), lambda i: (i, 0)),
            pl.BlockSpec((1, 256), lambda i: (0, 0)),
            pl.BlockSpec((1, 256), lambda i: (0, 0)),
        ],
        out_specs=[
            pl.BlockSpec((bm, 256), lambda i: (i, 0)),
            pl.BlockSpec((2, 256), lambda i: (0, 0)),
        ],
        out_shape=[
            jax.ShapeDtypeStruct((M, 256), jnp.float32),
            jax.ShapeDtypeStruct((2, 256), jnp.float32),
        ],
        scratch_shapes=[pltpu.VMEM((2, 256), jnp.float32)],
        compiler_params=pltpu.CompilerParams(
            dimension_semantics=("arbitrary",),
        ),
    )(ax.reshape(M, 1), w0.reshape(1, 256), b0.reshape(1, 256))


def _affine_from_stats(st, g, be):
    mu = st[0] / _N
    var = st[1] / _N - mu * mu
    al = g * lax.rsqrt(var + _EPS)
    return al, be - mu * al


def kernel(x, edge_index, edge_weights, W0, b0, g0, be0, W1, b1, g1, be1,
           W2, b2, g2, be2, Wf1, bf1, Wf2, bf2, Wo, bo):
    src = edge_index[0]
    dst = edge_index[1]

    (bsrc, bdl, bew, cnts, degp,
     bsrc2, bdl2, bew2, cnts2) = _run_pa(src, dst, edge_weights)
    bsrc3 = bsrc.reshape(_NW, _NCHUNK, _CAP)
    bdl3 = bdl.reshape(_NW, _NCHUNK, _CAP)
    bew3 = bew.reshape(_NW, _NCHUNK, _CAP)

    deg = degp.reshape(_NW, _NCHUNK * _CHB).sum(axis=0)[:_N] + 1.0
    dinv = lax.rsqrt(deg)
    y = dinv * x[:, 0]

    pyp = _run_pb(y, bsrc3, bdl3, bew3)
    sy = pyp.reshape(_NW, _NCHUNK * _CHB).sum(axis=0)[:_N]
    ax = dinv * (sy + y)

    r1, st1 = _l1_relu_stats(ax, W0[0], b0)
    al1, bt1 = _affine_from_stats(st1, g0, be0)

    G2 = _matmul(r1, W1, jnp.zeros((256,), jnp.float32), bm=3968, bn=256,
                 bk=256, affine=(al1, bt1), dinv=dinv)
    s2 = _run_pc(G2, bsrc2, bdl2, bew2, cnts2)
    r2, st2 = _relu_stats(s2, G2, dinv, b1)
    al2, bt2 = _affine_from_stats(st2, g1, be1)

    G3 = _matmul(r2, W2, jnp.zeros((256,), jnp.float32), bm=3968, bn=256,
                 bk=256, affine=(al2, bt2), dinv=dinv)
    s3 = _run_pc(G3, bsrc2, bdl2, bew2, cnts2)
    r3, st3 = _relu_stats(s3, G3, dinv, b2)
    al3, bt3 = _affine_from_stats(st3, g2, be2)

    h = r3.reshape(_B, _NPB * 256)
    alf = jnp.tile(al3, _NPB)
    btf = jnp.tile(bt3, _NPB)
    h = _matmul(h, Wf1, bf1, bm=_B, bn=1024, bk=512, act=True,
                affine=(alf, btf))
    h = _matmul(h, Wf2, bf2, bm=_B, bn=1024, bk=512, act=True)
    return _matmul(h, Wo, bo, bm=_B, bn=256, bk=512)
